# baseline (speedup 1.0000x reference)
import functools

import jax
import jax.numpy as jnp
from jax import lax
from jax.experimental import pallas as pl
from jax.experimental.pallas import tpu as pltpu
from jax.experimental.pallas import tpu_sc as plsc

N_NODES = 10000
N_EDGES = 320000
HIDDEN = 128
MSG_DIM = 16
BATCH = 2048
T_MAX = 1000.0

NC, NS = 2, 16
NW = NC * NS
E_PER_W = N_EDGES // NW
CHUNK = 80
N_CHUNKS = E_PER_W // CHUNK
ZROW = N_NODES
N_PAD = N_NODES + 8
B_PER_TILE = BATCH // NS
N_ACC = 10240
ROWS_PER_TILE = N_ACC // NS

BE = 3200


def _edge_feat_body(t_ref, src_ref, msg_ref, w_ref, wm_ref, bm_ref,
                    f_ref, srcm_ref):
    t = t_ref[...]
    mask = t <= T_MAX
    f = (jnp.cos(t * w_ref[...])
         + jnp.dot(msg_ref[...], wm_ref[...],
                   preferred_element_type=jnp.float32)
         + bm_ref[...])
    f_ref[...] = jnp.where(mask, f, 0.0)
    srcm_ref[...] = jnp.where(mask, src_ref[...], ZROW)


def _edge_features(t2, src2, msg, w_time, W_msg, bm):
    grid = N_EDGES // BE
    return pl.pallas_call(
        _edge_feat_body,
        grid=(grid,),
        in_specs=[
            pl.BlockSpec((BE, 1), lambda g: (g, 0)),
            pl.BlockSpec((BE, 1), lambda g: (g, 0)),
            pl.BlockSpec((BE, MSG_DIM), lambda g: (g, 0)),
            pl.BlockSpec((1, HIDDEN), lambda g: (0, 0)),
            pl.BlockSpec((MSG_DIM, HIDDEN), lambda g: (0, 0)),
            pl.BlockSpec((1, HIDDEN), lambda g: (0, 0)),
        ],
        out_specs=[
            pl.BlockSpec((BE, HIDDEN), lambda g: (g, 0)),
            pl.BlockSpec((BE, 1), lambda g: (g, 0)),
        ],
        out_shape=[
            jax.ShapeDtypeStruct((N_EDGES, HIDDEN), jnp.float32),
            jax.ShapeDtypeStruct((N_EDGES, 1), jnp.int32),
        ],
    )(t2, src2, msg, w_time, W_msg, bm)


def _sc_agg_body(emb_hbm, f_hbm, srcm_hbm, dst_hbm, idx_hbm, zeros_hbm,
                 sel_out, acc, srcv, dstv, rows, fbuf, idxv, sem):
    c = lax.axis_index("c")
    s = lax.axis_index("s")
    wid = c * NS + s
    pltpu.sync_copy(zeros_hbm.at[pl.ds(s * ROWS_PER_TILE, ROWS_PER_TILE), :],
                    acc.at[pl.ds(s * ROWS_PER_TILE, ROWS_PER_TILE), :])
    pltpu.sync_copy(srcm_hbm.at[wid], srcv)
    pltpu.sync_copy(dst_hbm.at[wid], dstv)
    plsc.subcore_barrier()

    base_e = wid * E_PER_W

    def body(k, carry):
        pltpu.async_copy(emb_hbm.at[srcv.at[k]], rows, sem).wait()
        pltpu.sync_copy(f_hbm.at[pl.ds(base_e + k * CHUNK, CHUNK), :], fbuf)
        pltpu.sync_copy(rows, acc.at[dstv.at[k]], add=True)
        pltpu.sync_copy(fbuf, acc.at[dstv.at[k]], add=True)
        return carry

    lax.fori_loop(0, N_CHUNKS, body, 0)
    plsc.subcore_barrier()

    pltpu.sync_copy(idx_hbm.at[pl.ds(s * B_PER_TILE, B_PER_TILE)], idxv)
    for j in range(2):
        half = rows.at[pl.ds(0, 64), :]
        pltpu.sync_copy(acc.at[idxv.at[pl.ds(j * 64, 64)]], half)
        pltpu.sync_copy(
            half, sel_out.at[c, pl.ds(s * B_PER_TILE + j * 64, 64), :])

    @pl.when(c == 0)
    def _():
        for j in range(2):
            half = fbuf.at[pl.ds(0, 64), :]
            pltpu.async_copy(emb_hbm.at[idxv.at[pl.ds(j * 64, 64)]],
                             half, sem).wait()
            pltpu.sync_copy(
                half, sel_out.at[2, pl.ds(s * B_PER_TILE + j * 64, 64), :])


@functools.cache
def _make_sc_agg():
    return functools.partial(
        pl.kernel,
        out_type=jax.ShapeDtypeStruct((3, BATCH, HIDDEN), jnp.float32),
        mesh=plsc.VectorSubcoreMesh(core_axis_name="c", subcore_axis_name="s",
                                    num_cores=NC, num_subcores=NS),
        compiler_params=pltpu.CompilerParams(use_tc_tiling_on_sc=False),
        scratch_types=[
            pltpu.VMEM_SHARED((N_ACC, HIDDEN), jnp.float32),
            pltpu.VMEM((N_CHUNKS, CHUNK), jnp.int32),
            pltpu.VMEM((N_CHUNKS, CHUNK), jnp.int32),
            pltpu.VMEM((CHUNK, HIDDEN), jnp.float32),
            pltpu.VMEM((CHUNK, HIDDEN), jnp.float32),
            pltpu.VMEM((B_PER_TILE,), jnp.int32),
            pltpu.SemaphoreType.DMA,
        ],
    )(_sc_agg_body)


def _cls_body(sel_ref, wu_ref, bu_ref, w1_ref, b1_ref, w2_ref, b2_ref,
              out_ref):
    x = sel_ref[0] + sel_ref[1] + sel_ref[2]
    h = jnp.maximum(
        jnp.dot(x, wu_ref[...], preferred_element_type=jnp.float32)
        + bu_ref[...], 0.0)
    h2 = jnp.maximum(
        jnp.dot(h, w1_ref[...], preferred_element_type=jnp.float32)
        + b1_ref[...], 0.0)
    out_ref[...] = (jnp.dot(h2, w2_ref[...],
                            preferred_element_type=jnp.float32)
                    + b2_ref[...])


def _classifier(sel, W_upd, bu, W1, b1, W2, b2):
    return pl.pallas_call(
        _cls_body,
        out_shape=jax.ShapeDtypeStruct((BATCH, HIDDEN), jnp.float32),
    )(sel, W_upd, bu, W1, b1, W2, b2)


def kernel(src, dst, t, msg, labels, idx, node_emb, w_time, W_msg, b_msg,
           W_upd, b_upd, W1, b1, W2, b2):
    del labels
    t2 = t.reshape(N_EDGES, 1)
    src2 = src.reshape(N_EDGES, 1)
    F, srcm = _edge_features(t2, src2, msg, w_time, W_msg,
                             b_msg.reshape(1, HIDDEN))
    emb_pad = jnp.concatenate(
        [node_emb, jnp.zeros((N_PAD - N_NODES, HIDDEN), node_emb.dtype)],
        axis=0)
    sel = _make_sc_agg()(emb_pad, F,
                  srcm.reshape(NW, N_CHUNKS, CHUNK),
                  dst.reshape(NW, N_CHUNKS, CHUNK),
                  idx,
                  jnp.zeros((N_ACC, HIDDEN), jnp.float32))
    return _classifier(sel, W_upd, b_upd.reshape(1, HIDDEN), W1,
                       b1.reshape(1, HIDDEN), W2, b2.reshape(1, HIDDEN))
